# Initial kernel scaffold; baseline (speedup 1.0000x reference)
import jax
import jax.numpy as jnp
from jax.experimental import pallas as pl


def kernel(emb, W1, b1, W2, b2, W3, b3, gamma, beta, Wl, bl, edge_index):
    raise NotImplementedError("write your pallas kernel here")


# speedup vs baseline: 1.8046x; 1.8046x over previous
import jax
import jax.numpy as jnp
from jax import lax
from jax.experimental import pallas as pl
from jax.experimental.pallas import tpu as pltpu
from jax.experimental.pallas import tpu_sc as plsc

N = 10000
D = 256
E = 160000
L = 40

NC = 2
NS = 16
NW = NC * NS

E_PAD = 163840
BLK = 2048
NBLK = E_PAD // BLK
GPB = BLK // 16

OWN = 320
N_PAD = NW * OWN
ACC_ROWS = 328
DUMP = 320
CBUF = 2176
RSIZE = 128
KD = D // 16

_MESH = plsc.VectorSubcoreMesh(
    core_axis_name="c", subcore_axis_name="s", num_cores=NC, num_subcores=NS)
_NOLAYOUT = pltpu.CompilerParams(needs_layout_passes=False)


def _sc_agg_body(src_hbm, dst_hbm, h_hbm, agg_out,
                 src_blk, dst_blk, csrc, cldst, gidx, sdx, rows, acc, deg,
                 rbuf, sem, do_norm=True):
  c = lax.axis_index("c")
  s = lax.axis_index("s")
  w = c * NS + s
  wlo = w * OWN

  z16 = jnp.zeros((16,), jnp.float32)
  zi16 = jnp.zeros((16,), jnp.int32)
  ones16 = jnp.ones((16,), jnp.float32)
  dump16 = jnp.full((16,), DUMP, jnp.int32)

  def zacc(i, carry):
    for k in range(KD):
      acc[i, pl.ds(k * 16, 16)] = z16
    return carry
  lax.fori_loop(0, ACC_ROWS, zacc, 0)
  def zdeg(i, carry):
    deg[pl.ds(i * 16, 16)] = z16
    return carry
  lax.fori_loop(0, ACC_ROWS // 8, zdeg, 0)

  def flush_round(r):
    for j in range(8):
      gidx[pl.ds(j * 16, 16)] = csrc[pl.ds(r * RSIZE + j * 16, 16)]
      sdx[pl.ds(j * 16, 16)] = cldst[pl.ds(r * RSIZE + j * 16, 16)]
    pltpu.async_copy(h_hbm.at[gidx], rows, sem).wait()
    for j in range(8):
      plsc.addupdate_scatter(deg, [sdx[pl.ds(j * 16, 16)]], ones16)
    def acc_grp(i16, carry):
      lvec = sdx[pl.ds(i16 * 16, 16)]
      base = i16 * 16
      for lane in range(16):
        r_own = lvec[lane]
        for k in range(KD):
          plsc.addupdate(acc.at[r_own, pl.ds(k * 16, 16)],
                         rows[base + lane, pl.ds(k * 16, 16)])
      return carry
    lax.fori_loop(0, 8, acc_grp, 0)

  def blk_body(b, cnt):
    eoff = b * BLK
    pltpu.sync_copy(src_hbm.at[pl.ds(eoff, BLK)], src_blk)
    pltpu.sync_copy(dst_hbm.at[pl.ds(eoff, BLK)], dst_blk)

    def grp(j, cnt2):
      dvec = dst_blk[pl.ds(j * 16, 16)]
      svec = src_blk[pl.ds(j * 16, 16)]
      m = (dvec >= wlo) & (dvec < wlo + OWN)
      plsc.store_compressed(csrc.at[pl.ds(cnt2, 16)], svec, mask=m)
      plsc.store_compressed(cldst.at[pl.ds(cnt2, 16)], dvec - wlo, mask=m)
      return cnt2 + jnp.sum(m.astype(jnp.int32))
    cnt = lax.fori_loop(0, GPB, grp, cnt)

    nr = cnt // RSIZE

    def fl(r, carry):
      flush_round(r)
      return carry
    lax.fori_loop(0, nr, fl, 0)

    tail = nr * RSIZE
    for j in range(8):
      csrc[pl.ds(j * 16, 16)] = csrc[pl.ds(tail + j * 16, 16)]
      cldst[pl.ds(j * 16, 16)] = cldst[pl.ds(tail + j * 16, 16)]
    return cnt - tail

  cnt = lax.fori_loop(0, NBLK, blk_body, 0)

  for j in range(8):
    csrc[pl.ds(cnt + j * 16, 16)] = zi16
    cldst[pl.ds(cnt + j * 16, 16)] = dump16
  flush_round(0)

  if do_norm:
    def norm(i16, carry):
      dv = jnp.maximum(deg[pl.ds(i16 * 16, 16)], 1.0)
      rinv = ones16 / dv
      for lane in range(16):
        r_own = i16 * 16 + lane
        dsp = rinv[lane] * ones16
        for k in range(KD):
          acc[r_own, pl.ds(k * 16, 16)] = acc[r_own, pl.ds(k * 16, 16)] * dsp
      return carry
    lax.fori_loop(0, OWN // 16, norm, 0)

  pltpu.sync_copy(acc.at[pl.ds(0, OWN)], agg_out.at[pl.ds(wlo, OWN)])


import functools as _ft

_sc_agg = pl.kernel(
    _sc_agg_body,
    out_type=(jax.ShapeDtypeStruct((N_PAD, D), jnp.float32),),
    mesh=_MESH,
    scratch_types=(
        pltpu.VMEM((BLK,), jnp.int32),
        pltpu.VMEM((BLK,), jnp.int32),
        pltpu.VMEM((CBUF,), jnp.int32),
        pltpu.VMEM((CBUF,), jnp.int32),
        pltpu.VMEM((RSIZE,), jnp.int32),
        pltpu.VMEM((RSIZE,), jnp.int32),
        pltpu.VMEM((RSIZE, D), jnp.float32),
        pltpu.VMEM((ACC_ROWS, D), jnp.float32),
        pltpu.VMEM((ACC_ROWS,), jnp.float32),
        pltpu.VMEM((16,), jnp.float32),
        pltpu.SemaphoreType.DMA,
    ),
    compiler_params=_NOLAYOUT)

_sc_agg_raw = pl.kernel(
    _ft.partial(_sc_agg_body, do_norm=False),
    out_type=(jax.ShapeDtypeStruct((N_PAD, D), jnp.float32),),
    mesh=_MESH,
    scratch_types=(
        pltpu.VMEM((BLK,), jnp.int32),
        pltpu.VMEM((BLK,), jnp.int32),
        pltpu.VMEM((CBUF,), jnp.int32),
        pltpu.VMEM((CBUF,), jnp.int32),
        pltpu.VMEM((RSIZE,), jnp.int32),
        pltpu.VMEM((RSIZE,), jnp.int32),
        pltpu.VMEM((RSIZE, D), jnp.float32),
        pltpu.VMEM((ACC_ROWS, D), jnp.float32),
        pltpu.VMEM((ACC_ROWS,), jnp.float32),
        pltpu.VMEM((16,), jnp.float32),
        pltpu.SemaphoreType.DMA,
    ),
    compiler_params=_NOLAYOUT)


def _mm_body(x_ref, w_ref, o_ref):
  o_ref[...] = jnp.dot(x_ref[...], w_ref[...],
                       preferred_element_type=jnp.float32)


def _tc_matmul(x, w):
  return pl.pallas_call(
      _mm_body,
      out_shape=jax.ShapeDtypeStruct((x.shape[0], w.shape[1]), jnp.float32),
  )(x, w)


def _mid_body(agg_ref, b_ref, g_ref, be_ref, w_ref, h_ref):
  x = agg_ref[:N, :] + b_ref[...]
  mu = jnp.mean(x, axis=0, keepdims=True)
  var = jnp.mean((x - mu) ** 2, axis=0, keepdims=True)
  x = (x - mu) * lax.rsqrt(var + 1e-5) * g_ref[...] + be_ref[...]
  x = jnp.where(x > 0, x, jnp.exp(x) - 1.0)
  h_ref[...] = jnp.dot(x, w_ref[...], preferred_element_type=jnp.float32)


def _tc_mid(agg, b, gamma, beta, w):
  return pl.pallas_call(
      _mid_body,
      out_shape=jax.ShapeDtypeStruct((N, D), jnp.float32),
  )(agg, b, gamma, beta, w)


def _final_body(agg_ref, b_ref, wl_ref, bl_ref, x_ref, lg_ref):
  x = agg_ref[:N, :] + b_ref[...]
  x_ref[...] = x
  lg_ref[...] = (jnp.dot(x, wl_ref[...], preferred_element_type=jnp.float32)
                 + bl_ref[...])


def _tc_final(agg, b, wl, bl):
  return pl.pallas_call(
      _final_body,
      out_shape=(jax.ShapeDtypeStruct((N, D), jnp.float32),
                 jax.ShapeDtypeStruct((N, L), jnp.float32)),
  )(agg, b, wl, bl)


def kernel(emb, W1, b1, W2, b2, W3, b3, gamma, beta, Wl, bl, edge_index):
  src = edge_index[0].astype(jnp.int32)
  dst = edge_index[1].astype(jnp.int32)
  npad = E_PAD - src.shape[0]
  src_p = jnp.concatenate([src, jnp.zeros((npad,), jnp.int32)])
  dst_p = jnp.concatenate([dst, jnp.full((npad,), -1, jnp.int32)])

  h1 = _tc_matmul(emb, W1)
  (agg1,) = _sc_agg(src_p, dst_p, h1)
  h2 = _tc_mid(agg1, b1, gamma, beta, W2)
  (agg2,) = _sc_agg(src_p, dst_p, h2)
  h3 = _tc_mid(agg2, b2, gamma, beta, W3)
  (agg3,) = _sc_agg(src_p, dst_p, h3)
  x3, logits = _tc_final(agg3, b3, Wl, bl)
  return (x3, logits)
